# baseline (speedup 1.0000x reference)
import functools

import jax
import jax.numpy as jnp
from jax import lax
from jax.experimental import pallas as pl
from jax.experimental.pallas import tpu as pltpu
from jax.experimental.pallas import tpu_sc as plsc

_R = 1
_NBUF = 8
_AHEAD = 4
_CMAX = 128


@functools.lru_cache(maxsize=None)
def _build(batch, hist, n_vocab, d):
    info = plsc.get_sparse_core_info()
    num_cores, num_subcores = info.num_cores, info.num_subcores
    num_workers = num_cores * num_subcores
    rows_per_w = batch // num_workers
    assert batch % num_workers == 0
    assert rows_per_w % (_R * _NBUF) == 0
    n_slots = rows_per_w // _R
    n_outer = n_slots // _NBUF
    chunks = []
    off = 0
    while off < hist:
        c = min(_CMAX, hist - off)
        chunks.append((off, c))
        off += c

    mesh = plsc.VectorSubcoreMesh(core_axis_name="c", subcore_axis_name="s")

    @functools.partial(
        pl.kernel,
        mesh=mesh,
        out_type=jax.ShapeDtypeStruct((batch * hist, 2 * d), jnp.float32),
        scratch_types=[
            pltpu.VMEM((_NBUF, _R, hist), jnp.int32),
            pltpu.VMEM((_NBUF, _R * hist, d), jnp.float32),
            [pltpu.SemaphoreType.DMA] * _NBUF,
            [pltpu.SemaphoreType.DMA] * _NBUF,
        ],
        compiler_params=pltpu.CompilerParams(use_tc_tiling_on_sc=False),
    )
    def k(idx_hbm, tab_hbm, out_hbm, idx_v, rows_v, gsem, osem):
        wid = lax.axis_index("s") * num_cores + lax.axis_index("c")
        base_b = wid * rows_per_w

        def fire(slot, g):
            b0 = base_b + g * _R
            pltpu.sync_copy(idx_hbm.at[pl.ds(b0, _R)], idx_v.at[slot])
            for r in range(_R):
                for off, c in chunks:
                    pltpu.async_copy(
                        tab_hbm.at[idx_v.at[slot, r, pl.ds(off, c)]],
                        rows_v.at[slot, pl.ds(r * hist + off, c)],
                        gsem[slot],
                    )

        def drain(slot):
            for r in range(_R):
                for off, c in chunks:
                    pltpu.make_async_copy(
                        tab_hbm.at[idx_v.at[slot, r, pl.ds(off, c)]],
                        rows_v.at[slot, pl.ds(r * hist + off, c)],
                        gsem[slot],
                    ).wait()

        def out_copy(slot, g):
            pltpu.async_copy(
                rows_v.at[slot],
                out_hbm.at[pl.ds((base_b + g * _R) * hist, _R * hist), pl.ds(0, d)],
                osem[slot],
            )

        def wait_out(slot, g):
            pltpu.make_async_copy(
                rows_v.at[slot],
                out_hbm.at[pl.ds((base_b + g * _R) * hist, _R * hist), pl.ds(0, d)],
                osem[slot],
            ).wait()

        for b in range(_AHEAD):
            fire(b, b)

        tail = _NBUF - _AHEAD

        def outer(go, _):
            for b in range(_NBUF):
                g = go * _NBUF + b
                fslot = (b + _AHEAD) % _NBUF
                if b < tail:
                    @pl.when(go >= 1)
                    def _():
                        wait_out(fslot, g + _AHEAD - _NBUF)
                else:
                    wait_out(fslot, g + _AHEAD - _NBUF)
                if b < tail:
                    fire(fslot, g + _AHEAD)
                else:
                    @pl.when(go < n_outer - 1)
                    def _():
                        fire(fslot, g + _AHEAD)
                drain(b)
                out_copy(b, g)
            return ()

        lax.fori_loop(0, n_outer, outer, ())

        for i in range(tail):
            g_last = n_slots - tail + i
            wait_out(g_last % _NBUF, g_last)

    return k


def kernel(indices, table):
    b, h = indices.shape
    v, d = table.shape
    out = _build(b, h, v, d)(indices.astype(jnp.int32), table)
    return out[:, :d].reshape(b, h, d)


# speedup vs baseline: 1.0144x; 1.0144x over previous
import functools

import jax
import jax.numpy as jnp
from jax import lax
from jax.experimental import pallas as pl
from jax.experimental.pallas import tpu as pltpu
from jax.experimental.pallas import tpu_sc as plsc

_SROWS = 2
_NBUF = 5
_AHEAD = 3
_TL = 8
_TB = 128


@functools.lru_cache(maxsize=None)
def _build(batch, hist, n_vocab, d):
    info = plsc.get_sparse_core_info()
    num_cores, num_subcores = info.num_cores, info.num_subcores
    num_workers = num_cores * num_subcores
    n_lb = hist // _TL
    n_bb = batch // _TB
    bb_per_w = n_bb // num_workers
    assert hist % _TL == 0 and batch % _TB == 0 and n_bb % num_workers == 0
    assert _TL % _SROWS == 0
    subs = _TL // _SROWS
    n_slots = n_lb * bb_per_w * subs
    assert n_slots % _NBUF == 0
    n_outer = n_slots // _NBUF

    mesh = plsc.VectorSubcoreMesh(core_axis_name="c", subcore_axis_name="s")

    @functools.partial(
        pl.kernel,
        mesh=mesh,
        out_type=jax.ShapeDtypeStruct((n_bb, _TB, hist, 2 * d), jnp.float32),
        scratch_types=[
            pltpu.VMEM((_NBUF, _SROWS, _TB), jnp.int32),
            pltpu.VMEM((_NBUF, _SROWS, _TB, d), jnp.float32),
            [pltpu.SemaphoreType.DMA] * _NBUF,
            [pltpu.SemaphoreType.DMA] * _NBUF,
        ],
        compiler_params=pltpu.CompilerParams(use_tc_tiling_on_sc=False),
    )
    def k(idx_hbm, tab_hbm, out_hbm, idx_v, rows_v, gsem, osem):
        wid = lax.axis_index("s") * num_cores + lax.axis_index("c")
        bb0 = wid * bb_per_w

        def decode(g):
            c = g // subs
            sub = g - c * subs
            lb = c // bb_per_w
            bb = bb0 + (c - lb * bb_per_w)
            return lb, bb, sub * _SROWS

        def fire(slot, g):
            lb, bb, s0 = decode(g)
            pltpu.sync_copy(idx_hbm.at[lb, bb, pl.ds(s0, _SROWS)], idx_v.at[slot])
            for r in range(_SROWS):
                pltpu.async_copy(
                    tab_hbm.at[idx_v.at[slot, r]],
                    rows_v.at[slot, r],
                    gsem[slot],
                )

        def drain(slot):
            for r in range(_SROWS):
                pltpu.make_async_copy(
                    tab_hbm.at[idx_v.at[slot, r]],
                    rows_v.at[slot, r],
                    gsem[slot],
                ).wait()

        def out_addr(g, r):
            lb, bb, s0 = decode(g)
            return out_hbm.at[bb, pl.ds(0, _TB), lb * _TL + s0 + r, pl.ds(0, d)]

        def out_copy(slot, g):
            for r in range(_SROWS):
                pltpu.async_copy(rows_v.at[slot, r], out_addr(g, r), osem[slot])

        def wait_out(slot, g):
            for r in range(_SROWS):
                pltpu.make_async_copy(
                    rows_v.at[slot, r], out_addr(g, r), osem[slot]
                ).wait()

        for b in range(_AHEAD):
            fire(b, b)

        tail = _NBUF - _AHEAD

        def outer(go, _):
            for b in range(_NBUF):
                g = go * _NBUF + b
                fslot = (b + _AHEAD) % _NBUF
                if b < tail:
                    @pl.when(go >= 1)
                    def _():
                        wait_out(fslot, g + _AHEAD - _NBUF)
                else:
                    wait_out(fslot, g + _AHEAD - _NBUF)
                if b < tail:
                    fire(fslot, g + _AHEAD)
                else:
                    @pl.when(go < n_outer - 1)
                    def _():
                        fire(fslot, g + _AHEAD)
                drain(b)
                out_copy(b, g)
            return ()

        lax.fori_loop(0, n_outer, outer, ())

        for i in range(tail):
            g_last = n_slots - tail + i
            wait_out(g_last % _NBUF, g_last)

    return k


def kernel(indices, table):
    b, h = indices.shape
    v, d = table.shape
    idx4 = (
        jnp.transpose(indices.astype(jnp.int32))
        .reshape(h // _TL, _TL, b // _TB, _TB)
        .transpose(0, 2, 1, 3)
    )
    out = _build(b, h, v, d)(idx4, table)
    return out[:, :, :, :d].reshape(b, h, d)
